# Initial kernel scaffold; baseline (speedup 1.0000x reference)
import jax
import jax.numpy as jnp
from jax.experimental import pallas as pl


def kernel(x, edge_index_rel1, edge_index_rel2, W1_r1, W1_r2, W2_r1, W2_r2, W3_r1, W3_r2, b1_r1, b1_r2, b2_r1, b2_r2, b3_r1, b3_r2):
    raise NotImplementedError("write your pallas kernel here")


# speedup vs baseline: 2.8084x; 2.8084x over previous
import dataclasses
import functools

import jax
import jax.numpy as jnp
from jax import lax
from jax.experimental import pallas as pl
from jax.experimental.pallas import tpu as pltpu
from jax.experimental.pallas import tpu_sc as plsc

N = 10000
N2 = 10240
E = 320000
D = 128
TILES = 16
PER_TILE = E // TILES
CHUNK = 128
SUPER = 16
NCHUNK = 160
NSUPER = NCHUNK // SUPER
PAD = NCHUNK * CHUNK - PER_TILE
DUMMY = N
RB = 2048
GRID_R = N2 // RB

_mesh = plsc.VectorSubcoreMesh(core_axis_name="c", subcore_axis_name="s")

_no_layout_cp = pltpu.CompilerParams()
if "needs_layout_passes" in pltpu.CompilerParams.__dataclass_fields__:
    _no_layout_cp = dataclasses.replace(_no_layout_cp, needs_layout_passes=False)


@functools.partial(
    pl.kernel,
    out_type=jax.ShapeDtypeStruct((64, N2), jnp.float32),
    mesh=_mesh,
    scratch_types=[
        pltpu.VMEM((PER_TILE,), jnp.int32),
        pltpu.VMEM((N2,), jnp.float32),
    ],
    compiler_params=_no_layout_cp,
)
def _degree_kernel(idx_hbm, out_hbm, idx_v, hist_v):
    c = lax.axis_index("c")
    s = lax.axis_index("s")
    zeros16 = jnp.zeros((16,), jnp.float32)
    ones16 = jnp.ones((16,), jnp.float32)
    for which in range(2):
        row = (c * 2 + which) * 16 + s
        pltpu.sync_copy(idx_hbm.at[row], idx_v)

        @pl.loop(0, N2, step=16)
        def _(i):
            hist_v.at[pl.ds(i, 16)][...] = zeros16

        @pl.loop(0, PER_TILE, step=16)
        def _(k):
            iv = idx_v[pl.ds(k, 16)]
            plsc.addupdate_scatter(hist_v, [iv], ones16)

        pltpu.sync_copy(hist_v, out_hbm.at[row])


@functools.partial(
    pl.kernel,
    out_type=jax.ShapeDtypeStruct((2 * N2, D), jnp.float32),
    mesh=_mesh,
    scratch_types=[
        pltpu.VMEM((SUPER, CHUNK), jnp.int32),
        pltpu.VMEM((SUPER, CHUNK), jnp.int32),
        pltpu.VMEM((CHUNK, D), jnp.float32),
        pltpu.VMEM_SHARED((N2, D), jnp.float32),
    ],
)
def _agg_kernel(g_hbm, idx_hbm, out_hbm, src_v, dst_v, rows_v, acc_sh):
    c = lax.axis_index("c")
    s = lax.axis_index("s")
    zeros16 = jnp.zeros((16,), jnp.float32)
    rows_per_tile = N2 // TILES

    @pl.loop(0, CHUNK)
    def _(r):
        @pl.loop(0, D, step=16)
        def _(j):
            rows_v.at[r, pl.ds(j, 16)][...] = zeros16

    @pl.loop(0, rows_per_tile, step=CHUNK)
    def _(i):
        pltpu.sync_copy(rows_v, acc_sh.at[pl.ds(s * rows_per_tile + i, CHUNK)])

    plsc.subcore_barrier()

    @pl.loop(0, NSUPER)
    def _(ss):
        pltpu.sync_copy(idx_hbm.at[c * 32 + s, pl.ds(ss * SUPER, SUPER)],
                        src_v)
        pltpu.sync_copy(idx_hbm.at[c * 32 + 16 + s, pl.ds(ss * SUPER, SUPER)],
                        dst_v)

        @pl.loop(0, SUPER)
        def _(j):
            pltpu.sync_copy(g_hbm.at[src_v.at[j]], rows_v)
            pltpu.sync_copy(rows_v, acc_sh.at[dst_v.at[j]], add=True)

    plsc.subcore_barrier()
    pltpu.sync_copy(
        acc_sh.at[pl.ds(s * rows_per_tile, rows_per_tile)],
        out_hbm.at[pl.ds(c * N2 + s * rows_per_tile, rows_per_tile)],
    )


def _scales_body(cnt_ref, out_ref):
    cnt = cnt_ref[...]
    s1 = jnp.sum(cnt[0:16], axis=0)
    d1 = jnp.sum(cnt[16:32], axis=0)
    s2 = jnp.sum(cnt[32:48], axis=0)
    d2 = jnp.sum(cnt[48:64], axis=0)
    st = jnp.stack([s1, s2, d1, d2])
    out_ref[...] = lax.rsqrt(jnp.maximum(st, 1.0))


def _scales_call(counts):
    return pl.pallas_call(
        _scales_body,
        grid=(GRID_R,),
        in_specs=[pl.BlockSpec((64, RB), lambda i: (0, i))],
        out_specs=pl.BlockSpec((4, RB), lambda i: (0, i)),
        out_shape=jax.ShapeDtypeStruct((4, N2), jnp.float32),
    )(counts)


def _prescale_body(x_ref, sc_ref, out_ref):
    r = pl.program_id(1)
    dso = jnp.where(r == 0, sc_ref[0], sc_ref[1])
    out_ref[...] = x_ref[...] * dso[:, None]


def _prescale_call(x, scales):
    return pl.pallas_call(
        _prescale_body,
        grid=(GRID_R, 2),
        in_specs=[
            pl.BlockSpec((RB, D), lambda i, r: (i, 0)),
            pl.BlockSpec((4, RB), lambda i, r: (0, i)),
        ],
        out_specs=pl.BlockSpec((RB, D), lambda i, r: (r * GRID_R + i, 0)),
        out_shape=jax.ShapeDtypeStruct((2 * N2, D), jnp.float32),
    )(x, scales)


def _layer_body(a1_ref, a2_ref, sc_ref, w1_ref, w2_ref, b1_ref, b2_ref,
                out_ref, *, relu, prescale):
    a1 = a1_ref[...] * sc_ref[2][:, None]
    a2 = a2_ref[...] * sc_ref[3][:, None]
    h = jnp.dot(a1, w1_ref[...], preferred_element_type=jnp.float32)
    h = h + jnp.dot(a2, w2_ref[...], preferred_element_type=jnp.float32)
    h = h + b1_ref[...] + b2_ref[...]
    if relu:
        h = jnp.maximum(h, 0.0)
    if prescale:
        r = pl.program_id(1)
        dso = jnp.where(r == 0, sc_ref[0], sc_ref[1])
        h = h * dso[:, None]
    out_ref[...] = h


def _layer_call(agg, scales, w1, w2, b1, b2):
    return pl.pallas_call(
        functools.partial(_layer_body, relu=True, prescale=True),
        grid=(GRID_R, 2),
        in_specs=[
            pl.BlockSpec((RB, D), lambda i, r: (i, 0)),
            pl.BlockSpec((RB, D), lambda i, r: (i + GRID_R, 0)),
            pl.BlockSpec((4, RB), lambda i, r: (0, i)),
            pl.BlockSpec((D, D), lambda i, r: (0, 0)),
            pl.BlockSpec((D, D), lambda i, r: (0, 0)),
            pl.BlockSpec((1, D), lambda i, r: (0, 0)),
            pl.BlockSpec((1, D), lambda i, r: (0, 0)),
        ],
        out_specs=pl.BlockSpec((RB, D), lambda i, r: (r * GRID_R + i, 0)),
        out_shape=jax.ShapeDtypeStruct((2 * N2, D), jnp.float32),
    )(agg, agg, scales, w1, w2, b1, b2)


def _final_call(agg, scales, w1, w2, b1, b2):
    return pl.pallas_call(
        functools.partial(_layer_body, relu=False, prescale=False),
        grid=(GRID_R,),
        in_specs=[
            pl.BlockSpec((RB, D), lambda i: (i, 0)),
            pl.BlockSpec((RB, D), lambda i: (i + GRID_R, 0)),
            pl.BlockSpec((4, RB), lambda i: (0, i)),
            pl.BlockSpec((D, D), lambda i: (0, 0)),
            pl.BlockSpec((D, D), lambda i: (0, 0)),
            pl.BlockSpec((1, D), lambda i: (0, 0)),
            pl.BlockSpec((1, D), lambda i: (0, 0)),
        ],
        out_specs=pl.BlockSpec((RB, D), lambda i: (i, 0)),
        out_shape=jax.ShapeDtypeStruct((N2, D), jnp.float32),
    )(agg, agg, scales, w1, w2, b1, b2)


def _pack_agg_idx(e1, e2):
    parts = []
    for rel, e in ((0, e1), (1, e2)):
        src = (e[0] + rel * N2).reshape(TILES, PER_TILE)
        dst = e[1].reshape(TILES, PER_TILE)
        src = jnp.pad(src, ((0, 0), (0, PAD)), constant_values=0)
        dst = jnp.pad(dst, ((0, 0), (0, PAD)), constant_values=DUMMY)
        parts.append(src.reshape(TILES, NCHUNK, CHUNK))
        parts.append(dst.reshape(TILES, NCHUNK, CHUNK))
    return jnp.stack(parts).reshape(64, NCHUNK, CHUNK)


def _pack_deg_idx(e1, e2):
    parts = [e1[0], e1[1], e2[0], e2[1]]
    return jnp.stack([p.reshape(TILES, PER_TILE) for p in parts]).reshape(
        64, PER_TILE)


def kernel(x, edge_index_rel1, edge_index_rel2,
           W1_r1, W1_r2, W2_r1, W2_r2, W3_r1, W3_r2,
           b1_r1, b1_r2, b2_r1, b2_r2, b3_r1, b3_r2):
    agg_idx = _pack_agg_idx(edge_index_rel1, edge_index_rel2)
    deg_idx = _pack_deg_idx(edge_index_rel1, edge_index_rel2)
    x_pad = jnp.pad(x, ((0, N2 - N), (0, 0)))

    counts = _degree_kernel(deg_idx)
    scales = _scales_call(counts)

    g = _prescale_call(x_pad, scales)
    agg = _agg_kernel(g, agg_idx)
    g = _layer_call(agg, scales, W1_r1, W1_r2,
                    b1_r1.reshape(1, D), b1_r2.reshape(1, D))
    agg = _agg_kernel(g, agg_idx)
    g = _layer_call(agg, scales, W2_r1, W2_r2,
                    b2_r1.reshape(1, D), b2_r2.reshape(1, D))
    agg = _agg_kernel(g, agg_idx)
    out = _final_call(agg, scales, W3_r1, W3_r2,
                      b3_r1.reshape(1, D), b3_r2.reshape(1, D))
    return out[:N]


# speedup vs baseline: 3.1163x; 1.1097x over previous
import dataclasses
import functools

import jax
import jax.numpy as jnp
from jax import lax
from jax.experimental import pallas as pl
from jax.experimental.pallas import tpu as pltpu
from jax.experimental.pallas import tpu_sc as plsc

N = 10000
N2 = 10240
E = 320000
D = 128
TILES = 16
PER_TILE = E // TILES
CHUNK = 128
SUPER = 16
NCHUNK = 160
NSUPER = NCHUNK // SUPER
PAD = NCHUNK * CHUNK - PER_TILE
DUMMY = N
RB = 2048
GRID_R = N2 // RB

_mesh = plsc.VectorSubcoreMesh(core_axis_name="c", subcore_axis_name="s")

_no_layout_cp = pltpu.CompilerParams()
if "needs_layout_passes" in pltpu.CompilerParams.__dataclass_fields__:
    _no_layout_cp = dataclasses.replace(_no_layout_cp, needs_layout_passes=False)


@functools.partial(
    pl.kernel,
    out_type=jax.ShapeDtypeStruct((64, N2), jnp.float32),
    mesh=_mesh,
    scratch_types=[
        pltpu.VMEM((PER_TILE,), jnp.int32),
        pltpu.VMEM((N2,), jnp.float32),
    ],
    compiler_params=_no_layout_cp,
)
def _degree_kernel(idx_hbm, out_hbm, idx_v, hist_v):
    c = lax.axis_index("c")
    s = lax.axis_index("s")
    zeros16 = jnp.zeros((16,), jnp.float32)
    ones16 = jnp.ones((16,), jnp.float32)
    for which in range(2):
        row = (c * 2 + which) * 16 + s
        pltpu.sync_copy(idx_hbm.at[row], idx_v)

        @pl.loop(0, N2, step=16)
        def _(i):
            hist_v.at[pl.ds(i, 16)][...] = zeros16

        @pl.loop(0, PER_TILE, step=16)
        def _(k):
            iv = idx_v[pl.ds(k, 16)]
            plsc.addupdate_scatter(hist_v, [iv], ones16)

        pltpu.sync_copy(hist_v, out_hbm.at[row])


@functools.partial(
    pl.kernel,
    out_type=jax.ShapeDtypeStruct((2 * N2, D), jnp.float32),
    mesh=_mesh,
    scratch_types=[
        pltpu.VMEM((SUPER, CHUNK), jnp.int32),
        pltpu.VMEM((SUPER, CHUNK), jnp.int32),
        pltpu.VMEM((SUPER, CHUNK), jnp.int32),
        pltpu.VMEM((SUPER, CHUNK), jnp.int32),
        pltpu.VMEM((CHUNK, D), jnp.float32),
        pltpu.VMEM((CHUNK, D), jnp.float32),
        pltpu.VMEM_SHARED((N2, D), jnp.float32),
        pltpu.SemaphoreType.DMA,
        pltpu.SemaphoreType.DMA,
        pltpu.SemaphoreType.DMA,
        pltpu.SemaphoreType.DMA,
        pltpu.SemaphoreType.DMA,
    ],
)
def _agg_kernel(g_hbm, idx_hbm, out_hbm, src0_v, dst0_v, src1_v, dst1_v,
                r0_v, r1_v, acc_sh, gs0, gs1, ss0, ss1, isem):
    c = lax.axis_index("c")
    s = lax.axis_index("s")
    zeros16 = jnp.zeros((16,), jnp.float32)
    rows_per_tile = N2 // TILES
    rows = (r0_v, r1_v)
    gsems = (gs0, gs1)
    ssems = (ss0, ss1)
    idxs = ((src0_v, dst0_v), (src1_v, dst1_v))

    @pl.loop(0, CHUNK)
    def _(r):
        @pl.loop(0, D, step=16)
        def _(j):
            r0_v.at[r, pl.ds(j, 16)][...] = zeros16

    @pl.loop(0, rows_per_tile, step=CHUNK)
    def _(i):
        pltpu.sync_copy(r0_v, acc_sh.at[pl.ds(s * rows_per_tile + i, CHUNK)])

    plsc.subcore_barrier()

    srow = c * 32 + s
    drow = c * 32 + 16 + s

    def stage(block_id, parity, sync):
        sv, dv = idxs[parity]
        if sync:
            pltpu.sync_copy(idx_hbm.at[srow, pl.ds(block_id * SUPER, SUPER)],
                            sv)
            pltpu.sync_copy(idx_hbm.at[drow, pl.ds(block_id * SUPER, SUPER)],
                            dv)
        else:
            pltpu.async_copy(
                idx_hbm.at[srow, pl.ds(block_id * SUPER, SUPER)], sv, isem)
            pltpu.async_copy(
                idx_hbm.at[drow, pl.ds(block_id * SUPER, SUPER)], dv, isem)

    def wait_stage(parity):
        sv, dv = idxs[parity]
        pltpu.make_async_copy(idx_hbm.at[srow, pl.ds(0, SUPER)], sv,
                              isem).wait()
        pltpu.make_async_copy(idx_hbm.at[drow, pl.ds(0, SUPER)], dv,
                              isem).wait()

    def chunk(i, parity, skip_drain=False, stage_next=None):
        b = i % 2
        sv, dv = idxs[parity]
        if not skip_drain:
            pltpu.make_async_copy(rows[b], acc_sh.at[dv.at[i]],
                                  ssems[b]).wait()
        if stage_next is not None:
            stage(stage_next, 1 - parity, sync=False)
        pltpu.async_copy(g_hbm.at[sv.at[i]], rows[b], gsems[b])
        pltpu.make_async_copy(g_hbm.at[sv.at[i]], rows[b], gsems[b]).wait()
        pltpu.async_copy(rows[b], acc_sh.at[dv.at[i]], ssems[b], add=True)

    stage(0, 0, sync=True)
    chunk(0, 0, skip_drain=True)
    chunk(1, 0, skip_drain=True)
    chunk(2, 0, stage_next=1)
    for i in range(3, SUPER):
        chunk(i, 0)

    @pl.loop(0, (NSUPER - 2) // 2)
    def _(q):
        for p, par in ((1, 1), (2, 0)):
            blk = 2 * q + p
            wait_stage(par)
            chunk(0, par)
            chunk(1, par)
            chunk(2, par, stage_next=blk + 1)
            for i in range(3, SUPER):
                chunk(i, par)

    wait_stage(1)
    for i in range(SUPER):
        chunk(i, 1)

    pltpu.make_async_copy(r0_v, acc_sh.at[dst1_v.at[SUPER - 2]], ss0).wait()
    pltpu.make_async_copy(r1_v, acc_sh.at[dst1_v.at[SUPER - 1]], ss1).wait()
    plsc.subcore_barrier()
    pltpu.sync_copy(
        acc_sh.at[pl.ds(s * rows_per_tile, rows_per_tile)],
        out_hbm.at[pl.ds(c * N2 + s * rows_per_tile, rows_per_tile)],
    )


def _scales_body(cnt_ref, out_ref):
    cnt = cnt_ref[...]
    s1 = jnp.sum(cnt[0:16], axis=0)
    d1 = jnp.sum(cnt[16:32], axis=0)
    s2 = jnp.sum(cnt[32:48], axis=0)
    d2 = jnp.sum(cnt[48:64], axis=0)
    st = jnp.stack([s1, s2, d1, d2])
    out_ref[...] = lax.rsqrt(jnp.maximum(st, 1.0))


def _scales_call(counts):
    return pl.pallas_call(
        _scales_body,
        grid=(GRID_R,),
        in_specs=[pl.BlockSpec((64, RB), lambda i: (0, i))],
        out_specs=pl.BlockSpec((4, RB), lambda i: (0, i)),
        out_shape=jax.ShapeDtypeStruct((4, N2), jnp.float32),
    )(counts)


def _prescale_body(x_ref, sc_ref, out_ref):
    r = pl.program_id(1)
    dso = jnp.where(r == 0, sc_ref[0], sc_ref[1])
    out_ref[...] = x_ref[...] * dso[:, None]


def _prescale_call(x, scales):
    return pl.pallas_call(
        _prescale_body,
        grid=(GRID_R, 2),
        in_specs=[
            pl.BlockSpec((RB, D), lambda i, r: (i, 0)),
            pl.BlockSpec((4, RB), lambda i, r: (0, i)),
        ],
        out_specs=pl.BlockSpec((RB, D), lambda i, r: (r * GRID_R + i, 0)),
        out_shape=jax.ShapeDtypeStruct((2 * N2, D), jnp.float32),
    )(x, scales)


def _layer_body(a1_ref, a2_ref, sc_ref, w1_ref, w2_ref, b1_ref, b2_ref,
                out_ref, *, relu, prescale):
    a1 = a1_ref[...] * sc_ref[2][:, None]
    a2 = a2_ref[...] * sc_ref[3][:, None]
    h = jnp.dot(a1, w1_ref[...], preferred_element_type=jnp.float32)
    h = h + jnp.dot(a2, w2_ref[...], preferred_element_type=jnp.float32)
    h = h + b1_ref[...] + b2_ref[...]
    if relu:
        h = jnp.maximum(h, 0.0)
    if prescale:
        r = pl.program_id(1)
        dso = jnp.where(r == 0, sc_ref[0], sc_ref[1])
        h = h * dso[:, None]
    out_ref[...] = h


def _layer_call(agg, scales, w1, w2, b1, b2):
    return pl.pallas_call(
        functools.partial(_layer_body, relu=True, prescale=True),
        grid=(GRID_R, 2),
        in_specs=[
            pl.BlockSpec((RB, D), lambda i, r: (i, 0)),
            pl.BlockSpec((RB, D), lambda i, r: (i + GRID_R, 0)),
            pl.BlockSpec((4, RB), lambda i, r: (0, i)),
            pl.BlockSpec((D, D), lambda i, r: (0, 0)),
            pl.BlockSpec((D, D), lambda i, r: (0, 0)),
            pl.BlockSpec((1, D), lambda i, r: (0, 0)),
            pl.BlockSpec((1, D), lambda i, r: (0, 0)),
        ],
        out_specs=pl.BlockSpec((RB, D), lambda i, r: (r * GRID_R + i, 0)),
        out_shape=jax.ShapeDtypeStruct((2 * N2, D), jnp.float32),
    )(agg, agg, scales, w1, w2, b1, b2)


def _final_call(agg, scales, w1, w2, b1, b2):
    return pl.pallas_call(
        functools.partial(_layer_body, relu=False, prescale=False),
        grid=(GRID_R,),
        in_specs=[
            pl.BlockSpec((RB, D), lambda i: (i, 0)),
            pl.BlockSpec((RB, D), lambda i: (i + GRID_R, 0)),
            pl.BlockSpec((4, RB), lambda i: (0, i)),
            pl.BlockSpec((D, D), lambda i: (0, 0)),
            pl.BlockSpec((D, D), lambda i: (0, 0)),
            pl.BlockSpec((1, D), lambda i: (0, 0)),
            pl.BlockSpec((1, D), lambda i: (0, 0)),
        ],
        out_specs=pl.BlockSpec((RB, D), lambda i: (i, 0)),
        out_shape=jax.ShapeDtypeStruct((N2, D), jnp.float32),
    )(agg, agg, scales, w1, w2, b1, b2)


def _pack_agg_idx(e1, e2):
    parts = []
    for rel, e in ((0, e1), (1, e2)):
        src = (e[0] + rel * N2).reshape(TILES, PER_TILE)
        dst = e[1].reshape(TILES, PER_TILE)
        src = jnp.pad(src, ((0, 0), (0, PAD)), constant_values=0)
        dst = jnp.pad(dst, ((0, 0), (0, PAD)), constant_values=DUMMY)
        parts.append(src.reshape(TILES, NCHUNK, CHUNK))
        parts.append(dst.reshape(TILES, NCHUNK, CHUNK))
    return jnp.stack(parts).reshape(64, NCHUNK, CHUNK)


def _pack_deg_idx(e1, e2):
    parts = [e1[0], e1[1], e2[0], e2[1]]
    return jnp.stack([p.reshape(TILES, PER_TILE) for p in parts]).reshape(
        64, PER_TILE)


def kernel(x, edge_index_rel1, edge_index_rel2,
           W1_r1, W1_r2, W2_r1, W2_r2, W3_r1, W3_r2,
           b1_r1, b1_r2, b2_r1, b2_r2, b3_r1, b3_r2):
    agg_idx = _pack_agg_idx(edge_index_rel1, edge_index_rel2)
    deg_idx = _pack_deg_idx(edge_index_rel1, edge_index_rel2)
    x_pad = jnp.pad(x, ((0, N2 - N), (0, 0)))

    counts = _degree_kernel(deg_idx)
    scales = _scales_call(counts)

    g = _prescale_call(x_pad, scales)
    agg = _agg_kernel(g, agg_idx)
    g = _layer_call(agg, scales, W1_r1, W1_r2,
                    b1_r1.reshape(1, D), b1_r2.reshape(1, D))
    agg = _agg_kernel(g, agg_idx)
    g = _layer_call(agg, scales, W2_r1, W2_r2,
                    b2_r1.reshape(1, D), b2_r2.reshape(1, D))
    agg = _agg_kernel(g, agg_idx)
    out = _final_call(agg, scales, W3_r1, W3_r2,
                      b3_r1.reshape(1, D), b3_r2.reshape(1, D))
    return out[:N]


# speedup vs baseline: 3.3002x; 1.0590x over previous
import dataclasses
import functools

import jax
import jax.numpy as jnp
from jax import lax
from jax.experimental import pallas as pl
from jax.experimental.pallas import tpu as pltpu
from jax.experimental.pallas import tpu_sc as plsc

N = 10000
N2 = 10240
E = 320000
D = 128
TILES = 16
PER_TILE = E // TILES
CHUNK = 64
SUPER = 16
NCHUNK = 320
NSUPER = NCHUNK // SUPER
PAD = NCHUNK * CHUNK - PER_TILE
LAG = 2
KBUF = 4
DUMMY = N
RB = 2048
GRID_R = N2 // RB

_mesh = plsc.VectorSubcoreMesh(core_axis_name="c", subcore_axis_name="s")

_PROBE_SCATTER = False

_no_layout_cp = pltpu.CompilerParams()
if "needs_layout_passes" in pltpu.CompilerParams.__dataclass_fields__:
    _no_layout_cp = dataclasses.replace(_no_layout_cp, needs_layout_passes=False)


@functools.partial(
    pl.kernel,
    out_type=jax.ShapeDtypeStruct((64, N2), jnp.float32),
    mesh=_mesh,
    scratch_types=[
        pltpu.VMEM((PER_TILE,), jnp.int32),
        pltpu.VMEM((N2,), jnp.float32),
    ],
    compiler_params=_no_layout_cp,
)
def _degree_kernel(idx_hbm, out_hbm, idx_v, hist_v):
    c = lax.axis_index("c")
    s = lax.axis_index("s")
    zeros16 = jnp.zeros((16,), jnp.float32)
    ones16 = jnp.ones((16,), jnp.float32)
    for which in range(2):
        row = (c * 2 + which) * 16 + s
        pltpu.sync_copy(idx_hbm.at[row], idx_v)

        @pl.loop(0, N2, step=16)
        def _(i):
            hist_v.at[pl.ds(i, 16)][...] = zeros16

        @pl.loop(0, PER_TILE, step=16)
        def _(k):
            iv = idx_v[pl.ds(k, 16)]
            plsc.addupdate_scatter(hist_v, [iv], ones16)

        pltpu.sync_copy(hist_v, out_hbm.at[row])


@functools.partial(
    pl.kernel,
    out_type=jax.ShapeDtypeStruct((2 * N2, D), jnp.float32),
    mesh=_mesh,
    scratch_types=[
        pltpu.VMEM((SUPER, CHUNK), jnp.int32),
        pltpu.VMEM((SUPER, CHUNK), jnp.int32),
        pltpu.VMEM((SUPER, CHUNK), jnp.int32),
        pltpu.VMEM((SUPER, CHUNK), jnp.int32),
        [pltpu.VMEM((CHUNK, D), jnp.float32)] * KBUF,
        [pltpu.SemaphoreType.DMA] * KBUF,
        [pltpu.SemaphoreType.DMA] * KBUF,
        pltpu.VMEM_SHARED((N2, D), jnp.float32),
        pltpu.SemaphoreType.DMA,
    ],
)
def _agg_kernel(g_hbm, idx_hbm, out_hbm, src0_v, dst0_v, src1_v, dst1_v,
                rows, gsems, ssems, acc_sh, isem):
    c = lax.axis_index("c")
    s = lax.axis_index("s")
    zeros16 = jnp.zeros((16,), jnp.float32)
    rows_per_tile = N2 // TILES
    idxs = ((src0_v, dst0_v), (src1_v, dst1_v))

    @pl.loop(0, CHUNK)
    def _(r):
        @pl.loop(0, D, step=16)
        def _(j):
            rows[0].at[r, pl.ds(j, 16)][...] = zeros16

    @pl.loop(0, rows_per_tile, step=CHUNK)
    def _(i):
        pltpu.sync_copy(rows[0],
                        acc_sh.at[pl.ds(s * rows_per_tile + i, CHUNK)])

    plsc.subcore_barrier()

    srow = c * 32 + s
    drow = c * 32 + 16 + s

    def stage(block_id, parity, sync):
        sv, dv = idxs[parity]
        if sync:
            pltpu.sync_copy(idx_hbm.at[srow, pl.ds(block_id * SUPER, SUPER)],
                            sv)
            pltpu.sync_copy(idx_hbm.at[drow, pl.ds(block_id * SUPER, SUPER)],
                            dv)
        else:
            pltpu.async_copy(
                idx_hbm.at[srow, pl.ds(block_id * SUPER, SUPER)], sv, isem)
            pltpu.async_copy(
                idx_hbm.at[drow, pl.ds(block_id * SUPER, SUPER)], dv, isem)

    def wait_stage(parity):
        sv, dv = idxs[parity]
        pltpu.make_async_copy(idx_hbm.at[srow, pl.ds(0, SUPER)], sv,
                              isem).wait()
        pltpu.make_async_copy(idx_hbm.at[drow, pl.ds(0, SUPER)], dv,
                              isem).wait()

    def drain_scatter(b, dv):
        pltpu.make_async_copy(rows[b], acc_sh.at[dv.at[0]], ssems[b]).wait()

    def fire_scatter(j, dv):
        b = j % KBUF
        pltpu.make_async_copy(g_hbm.at[dv.at[j]], rows[b], gsems[b]).wait()
        pltpu.async_copy(rows[b], acc_sh.at[dv.at[j]], ssems[b], add=True)

    def block(parity, *, first=False, stage_next=None):
        sv, dv = idxs[parity]
        psv, pdv = idxs[1 - parity]
        if not first:
            wait_stage(parity)
        for i in range(SUPER):
            b = i % KBUF
            if not (first and i < KBUF):
                drain_scatter(b, dv)
            if i == KBUF and stage_next is not None:
                stage(stage_next, 1 - parity, sync=False)
            pltpu.async_copy(g_hbm.at[sv.at[i]], rows[b], gsems[b])
            j = i - LAG
            if j >= 0:
                fire_scatter(j, dv)
            elif not first:
                fire_scatter(SUPER + j, pdv)

    stage(0, 0, sync=True)
    block(0, first=True, stage_next=1)

    @pl.loop(0, (NSUPER - 2) // 2)
    def _(q):
        block(1, stage_next=2 * q + 2)
        block(0, stage_next=2 * q + 3)

    block(1)

    for j in range(SUPER - LAG, SUPER):
        fire_scatter(j, idxs[1][1])
    for b in range(KBUF):
        drain_scatter(b, idxs[1][1])
    plsc.subcore_barrier()
    pltpu.sync_copy(
        acc_sh.at[pl.ds(s * rows_per_tile, rows_per_tile)],
        out_hbm.at[pl.ds(c * N2 + s * rows_per_tile, rows_per_tile)],
    )


def _scales_body(cnt_ref, out_ref):
    cnt = cnt_ref[...]
    s1 = jnp.sum(cnt[0:16], axis=0)
    d1 = jnp.sum(cnt[16:32], axis=0)
    s2 = jnp.sum(cnt[32:48], axis=0)
    d2 = jnp.sum(cnt[48:64], axis=0)
    st = jnp.stack([s1, s2, d1, d2])
    out_ref[...] = lax.rsqrt(jnp.maximum(st, 1.0))


def _scales_call(counts):
    return pl.pallas_call(
        _scales_body,
        grid=(GRID_R,),
        in_specs=[pl.BlockSpec((64, RB), lambda i: (0, i))],
        out_specs=pl.BlockSpec((4, RB), lambda i: (0, i)),
        out_shape=jax.ShapeDtypeStruct((4, N2), jnp.float32),
    )(counts)


def _prescale_body(x_ref, sc_ref, out_ref):
    r = pl.program_id(1)
    dso = jnp.where(r == 0, sc_ref[0], sc_ref[1])
    out_ref[...] = x_ref[...] * dso[:, None]


def _prescale_call(x, scales):
    return pl.pallas_call(
        _prescale_body,
        grid=(GRID_R, 2),
        in_specs=[
            pl.BlockSpec((RB, D), lambda i, r: (i, 0)),
            pl.BlockSpec((4, RB), lambda i, r: (0, i)),
        ],
        out_specs=pl.BlockSpec((RB, D), lambda i, r: (r * GRID_R + i, 0)),
        out_shape=jax.ShapeDtypeStruct((2 * N2, D), jnp.float32),
    )(x, scales)


def _layer_body(a1_ref, a2_ref, sc_ref, w1_ref, w2_ref, b1_ref, b2_ref,
                out_ref, *, relu, prescale):
    a1 = a1_ref[...] * sc_ref[2][:, None]
    a2 = a2_ref[...] * sc_ref[3][:, None]
    h = jnp.dot(a1, w1_ref[...], preferred_element_type=jnp.float32)
    h = h + jnp.dot(a2, w2_ref[...], preferred_element_type=jnp.float32)
    h = h + b1_ref[...] + b2_ref[...]
    if relu:
        h = jnp.maximum(h, 0.0)
    if prescale:
        r = pl.program_id(1)
        dso = jnp.where(r == 0, sc_ref[0], sc_ref[1])
        h = h * dso[:, None]
    out_ref[...] = h


def _layer_call(agg, scales, w1, w2, b1, b2):
    return pl.pallas_call(
        functools.partial(_layer_body, relu=True, prescale=True),
        grid=(GRID_R, 2),
        in_specs=[
            pl.BlockSpec((RB, D), lambda i, r: (i, 0)),
            pl.BlockSpec((RB, D), lambda i, r: (i + GRID_R, 0)),
            pl.BlockSpec((4, RB), lambda i, r: (0, i)),
            pl.BlockSpec((D, D), lambda i, r: (0, 0)),
            pl.BlockSpec((D, D), lambda i, r: (0, 0)),
            pl.BlockSpec((1, D), lambda i, r: (0, 0)),
            pl.BlockSpec((1, D), lambda i, r: (0, 0)),
        ],
        out_specs=pl.BlockSpec((RB, D), lambda i, r: (r * GRID_R + i, 0)),
        out_shape=jax.ShapeDtypeStruct((2 * N2, D), jnp.float32),
    )(agg, agg, scales, w1, w2, b1, b2)


def _final_call(agg, scales, w1, w2, b1, b2):
    return pl.pallas_call(
        functools.partial(_layer_body, relu=False, prescale=False),
        grid=(GRID_R,),
        in_specs=[
            pl.BlockSpec((RB, D), lambda i: (i, 0)),
            pl.BlockSpec((RB, D), lambda i: (i + GRID_R, 0)),
            pl.BlockSpec((4, RB), lambda i: (0, i)),
            pl.BlockSpec((D, D), lambda i: (0, 0)),
            pl.BlockSpec((D, D), lambda i: (0, 0)),
            pl.BlockSpec((1, D), lambda i: (0, 0)),
            pl.BlockSpec((1, D), lambda i: (0, 0)),
        ],
        out_specs=pl.BlockSpec((RB, D), lambda i: (i, 0)),
        out_shape=jax.ShapeDtypeStruct((N2, D), jnp.float32),
    )(agg, agg, scales, w1, w2, b1, b2)


def _pack_agg_idx(e1, e2):
    parts = []
    for rel, e in ((0, e1), (1, e2)):
        src = (e[0] + rel * N2).reshape(TILES, PER_TILE)
        dst = e[1].reshape(TILES, PER_TILE)
        src = jnp.pad(src, ((0, 0), (0, PAD)), constant_values=0)
        dst = jnp.pad(dst, ((0, 0), (0, PAD)), constant_values=DUMMY)
        parts.append(src.reshape(TILES, NCHUNK, CHUNK))
        parts.append(dst.reshape(TILES, NCHUNK, CHUNK))
    return jnp.stack(parts).reshape(64, NCHUNK, CHUNK)


def _pack_deg_idx(e1, e2):
    parts = [e1[0], e1[1], e2[0], e2[1]]
    return jnp.stack([p.reshape(TILES, PER_TILE) for p in parts]).reshape(
        64, PER_TILE)


def kernel(x, edge_index_rel1, edge_index_rel2,
           W1_r1, W1_r2, W2_r1, W2_r2, W3_r1, W3_r2,
           b1_r1, b1_r2, b2_r1, b2_r2, b3_r1, b3_r2):
    agg_idx = _pack_agg_idx(edge_index_rel1, edge_index_rel2)
    deg_idx = _pack_deg_idx(edge_index_rel1, edge_index_rel2)
    x_pad = jnp.pad(x, ((0, N2 - N), (0, 0)))

    counts = _degree_kernel(deg_idx)
    scales = _scales_call(counts)

    g = _prescale_call(x_pad, scales)
    agg = _agg_kernel(g, agg_idx)
    g = _layer_call(agg, scales, W1_r1, W1_r2,
                    b1_r1.reshape(1, D), b1_r2.reshape(1, D))
    agg = _agg_kernel(g, agg_idx)
    g = _layer_call(agg, scales, W2_r1, W2_r2,
                    b2_r1.reshape(1, D), b2_r2.reshape(1, D))
    agg = _agg_kernel(g, agg_idx)
    out = _final_call(agg, scales, W3_r1, W3_r2,
                      b3_r1.reshape(1, D), b3_r2.reshape(1, D))
    return out[:N]
